# baseline (speedup 1.0000x reference)
import functools

import jax
import jax.numpy as jnp
from jax.experimental import pallas as pl
from jax.experimental.pallas import tpu as pltpu
from jax.experimental.pallas import tpu_sc as plsc
from jax import lax

_NEG = float("-inf")
_BIG = 2**30


def _top6(cv, ci):
    vs, js = [], []
    for _ in range(6):
        m = jnp.max(cv, axis=1, keepdims=True)
        isel = jnp.min(jnp.where(cv == m, ci, _BIG), axis=1, keepdims=True)
        vs.append(m)
        js.append(isel)
        cv = jnp.where((cv == m) & (ci == isel), _NEG, cv)
    return jnp.concatenate(vs, axis=1), jnp.concatenate(js, axis=1)


def _prep_body(feat_ref, w_ref, b_ref, tir_ref, tic_ref,
               so_ref, outf_ref, win_ref, disp_ref):
    f = feat_ref[...]
    B = f.shape[0]
    preds = jnp.dot(f, w_ref[...], preferred_element_type=jnp.float32)
    preds = preds + b_ref[...]
    m = jnp.max(preds, axis=1, keepdims=True)
    e = jnp.exp(preds - m)
    so = e / jnp.sum(e, axis=1, keepdims=True)
    so_ref[...] = so

    nrm = jnp.sqrt(jnp.sum(f * f, axis=1, keepdims=True))
    nrm = jnp.maximum(nrm, 1e-12)
    outf_ref[...] = f / nrm

    sv = jnp.sum(so, axis=0, keepdims=True)
    disp = (jnp.sum(sv * sv) - jnp.sum(so * so)) / float(B)
    disp_ref[...] = jnp.reshape(disp, (1, 1))

    tir = tir_ref[...]
    tic = tic_ref[...]
    row = jax.lax.broadcasted_iota(jnp.int32, (B, B), 0)
    col = jax.lax.broadcasted_iota(jnp.int32, (B, B), 1)
    eq = (tic == tir) & (row > col)
    loser = jnp.max(jnp.where(eq, 1, 0), axis=0, keepdims=True)
    win_ref[...] = 1 - loser


def _stream_body(outf_ref, bank_ref, mask_ref,
                 win_ref, tir_ref, vout_ref, iout_ref, rv_ref, ri_ref,
                 *, nb, bn):
    g = pl.program_id(0)

    @pl.when(g == 0)
    def _init():
        B = outf_ref.shape[0]
        rv_ref[...] = jnp.full((B, 8), _NEG, jnp.float32)
        ri_ref[...] = jnp.full((B, 8), -1, jnp.int32)

    q = outf_ref[...]
    B = q.shape[0]
    s = jax.lax.dot_general(q, bank_ref[...],
                            (((1,), (1,)), ((), ())),
                            preferred_element_type=jnp.float32)
    s = jnp.where(mask_ref[...] > 0, _NEG, s)

    coll = jax.lax.broadcasted_iota(jnp.int32, (B, bn), 1)
    bvs, bis = [], []
    for _ in range(6):
        m = jnp.max(s, axis=1, keepdims=True)
        a = jnp.argmax(s, axis=1, keepdims=True).astype(jnp.int32)
        bvs.append(m)
        bis.append(g * bn + a)
        s = jnp.where(coll == a, _NEG, s)
    bv = jnp.concatenate(bvs, axis=1)
    bi = jnp.concatenate(bis, axis=1)

    pad2v = jnp.full((B, 2), _NEG, jnp.float32)
    pad2i = jnp.full((B, 2), -1, jnp.int32)
    cv = jnp.concatenate([rv_ref[...], bv, pad2v], axis=1)
    ci = jnp.concatenate([ri_ref[...], bi, pad2i], axis=1)
    nv, ni = _top6(cv, ci)
    rv_ref[...] = jnp.concatenate([nv, pad2v], axis=1)
    ri_ref[...] = jnp.concatenate([ni, pad2i], axis=1)

    @pl.when(g == nb - 1)
    def _finish():
        gc = jax.lax.dot_general(q, q,
                                 (((1,), (1,)), ((), ())),
                                 preferred_element_type=jnp.float32)
        winb = win_ref[...] > 0
        v = jnp.where(winb, gc, _NEG)
        colb = jax.lax.broadcasted_iota(jnp.int32, (B, B), 1)
        tir = tir_ref[...]
        pvs, pis = [], []
        for _ in range(6):
            m = jnp.max(v, axis=1, keepdims=True)
            bsel = jnp.min(jnp.where(v == m, colb, _BIG), axis=1, keepdims=True)
            jsel = jnp.max(jnp.where(colb == bsel, tir, -1), axis=1,
                           keepdims=True)
            pvs.append(m)
            pis.append(jsel)
            v = jnp.where(colb == bsel, _NEG, v)
        cv2 = jnp.concatenate([nv, jnp.concatenate(pvs, axis=1), pad2v, pad2v],
                              axis=1)
        ci2 = jnp.concatenate([ni, jnp.concatenate(pis, axis=1), pad2i, pad2i],
                              axis=1)
        fv, fi = _top6(cv2, ci2)
        vout_ref[...] = jnp.concatenate([fv, pad2v], axis=1)
        iout_ref[...] = jnp.concatenate([fi, pad2i], axis=1)


def _kl_body(g1_ref, if5_ref, qx_ref, tir_ref, win_ref, so_ref, disp_ref,
             out_ref, acc_ref, *, nchunk, b):
    c = pl.program_id(0)

    @pl.when(c == 0)
    def _init():
        acc_ref[0, 0] = 0.0

    if5c = if5_ref[...]
    tir = tir_ref[...]
    so = so_ref[...]
    e = ((if5c == tir) & (win_ref[...] > 0)).astype(jnp.float32)
    patch = jnp.dot(e, so, preferred_element_type=jnp.float32)
    hasp = jnp.sum(e, axis=1, keepdims=True) > 0
    g1w = g1_ref[...]
    cc = g1w.shape[1] // 2
    odd = (if5c - (if5c // 2) * 2) == 1
    base = jnp.where(odd, g1w[:, cc:], g1w[:, :cc])
    rows = jnp.where(hasp, patch, base)
    B = tir.shape[1]
    ch = if5c.shape[0]
    colb = jax.lax.broadcasted_iota(jnp.int32, (ch, B), 1)
    qoh = (qx_ref[...] == colb).astype(jnp.float32)
    soq = jnp.dot(qoh, so, preferred_element_type=jnp.float32)
    acc_ref[0, 0] += jnp.sum(rows * (jnp.log(rows) - soq))

    @pl.when(c == nchunk - 1)
    def _finish():
        out_ref[...] = jnp.reshape(acc_ref[0, 0] / float(b) + disp_ref[0, 0],
                                   (1, 1))


def _make_sc_gather(n, d, btot):
    info = plsc.get_sparse_core_info()
    nc, ns = info.num_cores, info.num_subcores
    nw = nc * ns
    bpw = btot // nw
    mesh = plsc.VectorSubcoreMesh(core_axis_name="c", subcore_axis_name="s")

    @functools.partial(
        pl.kernel, mesh=mesh,
        out_type=jax.ShapeDtypeStruct((btot, d), jnp.float32),
        scratch_types=[
            pltpu.VMEM((bpw,), jnp.int32),
            pltpu.VMEM((bpw, d), jnp.float32),
            pltpu.SemaphoreType.DMA,
        ],
    )
    def _sc_gather(table_hbm, idx_hbm, out_hbm, idx_v, rows_v, sem):
        wid = lax.axis_index("s") * nc + lax.axis_index("c")
        base = wid * bpw
        pltpu.sync_copy(idx_hbm.at[pl.ds(base, bpw)], idx_v)
        pltpu.async_copy(table_hbm.at[idx_v], rows_v, sem).wait()
        pltpu.sync_copy(rows_v, out_hbm.at[pl.ds(base, bpw)])

    return _sc_gather


def kernel(features, fea_bank, W_cls, b_cls, score_bank, trg_idx):
    B, D = features.shape
    N = fea_bank.shape[0]
    C = W_cls.shape[1]
    K = 5
    ti = trg_idx.astype(jnp.int32)
    tir = ti.reshape(1, B)
    tic = ti.reshape(B, 1)

    so, outf, win, disp = pl.pallas_call(
        _prep_body,
        out_shape=[
            jax.ShapeDtypeStruct((B, C), jnp.float32),
            jax.ShapeDtypeStruct((B, D), jnp.float32),
            jax.ShapeDtypeStruct((1, B), jnp.int32),
            jax.ShapeDtypeStruct((1, 1), jnp.float32),
        ],
    )(features, W_cls, b_cls.reshape(1, C), tir, tic)

    BN = 2048
    NB = -(-N // BN)
    npad = NB * BN

    maskp = jnp.zeros((1, npad), jnp.float32)
    maskp = maskp.at[0, N:].set(1.0)
    maskp = maskp.at[0, ti].set(1.0)

    vals6, idx6 = pl.pallas_call(
        functools.partial(_stream_body, nb=NB, bn=BN),
        grid=(NB,),
        in_specs=[
            pl.BlockSpec((B, D), lambda g: (0, 0)),
            pl.BlockSpec((BN, D), lambda g: (g, 0)),
            pl.BlockSpec((1, BN), lambda g: (0, g)),
            pl.BlockSpec((1, B), lambda g: (0, 0)),
            pl.BlockSpec((1, B), lambda g: (0, 0)),
        ],
        out_specs=[
            pl.BlockSpec((B, 8), lambda g: (0, 0)),
            pl.BlockSpec((B, 8), lambda g: (0, 0)),
        ],
        out_shape=[
            jax.ShapeDtypeStruct((B, 8), jnp.float32),
            jax.ShapeDtypeStruct((B, 8), jnp.int32),
        ],
        scratch_shapes=[
            pltpu.VMEM((B, 8), jnp.float32),
            pltpu.VMEM((B, 8), jnp.int32),
        ],
    )(outf, fea_bank, maskp, win, tir)

    if5 = idx6[:, 1:1 + K].reshape(B * K, 1)
    if5_flat = if5.reshape(B * K)

    sbw = score_bank.reshape(N // 2, 2 * C)
    g1 = _make_sc_gather(N // 2, 2 * C, B * K)(sbw, if5_flat // 2)

    CH = 640
    NCH = B * K // CH
    qx = (jnp.arange(B * K, dtype=jnp.int32) // K).reshape(B * K, 1)
    loss = pl.pallas_call(
        functools.partial(_kl_body, nchunk=NCH, b=B),
        grid=(NCH,),
        in_specs=[
            pl.BlockSpec((CH, 2 * C), lambda c: (c, 0)),
            pl.BlockSpec((CH, 1), lambda c: (c, 0)),
            pl.BlockSpec((CH, 1), lambda c: (c, 0)),
            pl.BlockSpec((1, B), lambda c: (0, 0)),
            pl.BlockSpec((1, B), lambda c: (0, 0)),
            pl.BlockSpec((B, C), lambda c: (0, 0)),
            pl.BlockSpec((1, 1), lambda c: (0, 0)),
        ],
        out_specs=pl.BlockSpec((1, 1), lambda c: (0, 0)),
        out_shape=jax.ShapeDtypeStruct((1, 1), jnp.float32),
        scratch_shapes=[pltpu.SMEM((1, 1), jnp.float32)],
    )(g1, if5, qx, tir, win, so, disp)

    return loss[0, 0]


# speedup vs baseline: 1.0363x; 1.0363x over previous
import functools

import jax
import jax.numpy as jnp
from jax.experimental import pallas as pl
from jax.experimental.pallas import tpu as pltpu
from jax.experimental.pallas import tpu_sc as plsc
from jax import lax

_NEG = float("-inf")
_BIG = 2**30


def _top6(cv, ci):
    vs, js = [], []
    for _ in range(6):
        m = jnp.max(cv, axis=1, keepdims=True)
        isel = jnp.min(jnp.where(cv == m, ci, _BIG), axis=1, keepdims=True)
        vs.append(m)
        js.append(isel)
        cv = jnp.where((cv == m) & (ci == isel), _NEG, cv)
    return jnp.concatenate(vs, axis=1), jnp.concatenate(js, axis=1)


def _prep_body(feat_ref, w_ref, b_ref, tir_ref, tic_ref,
               so_ref, outf_ref, win_ref, disp_ref):
    f = feat_ref[...]
    B = f.shape[0]
    preds = jnp.dot(f, w_ref[...], preferred_element_type=jnp.float32)
    preds = preds + b_ref[...]
    m = jnp.max(preds, axis=1, keepdims=True)
    e = jnp.exp(preds - m)
    so = e / jnp.sum(e, axis=1, keepdims=True)
    so_ref[...] = so

    nrm = jnp.sqrt(jnp.sum(f * f, axis=1, keepdims=True))
    nrm = jnp.maximum(nrm, 1e-12)
    outf_ref[...] = f / nrm

    sv = jnp.sum(so, axis=0, keepdims=True)
    disp = (jnp.sum(sv * sv) - jnp.sum(so * so)) / float(B)
    disp_ref[...] = jnp.reshape(disp, (1, 1))

    tir = tir_ref[...]
    tic = tic_ref[...]
    row = jax.lax.broadcasted_iota(jnp.int32, (B, B), 0)
    col = jax.lax.broadcasted_iota(jnp.int32, (B, B), 1)
    eq = (tic == tir) & (row > col)
    loser = jnp.max(jnp.where(eq, 1, 0), axis=0, keepdims=True)
    win_ref[...] = 1 - loser


def _stream_body(outf_ref, bank_ref, mask_ref,
                 win_ref, tir_ref, vout_ref, iout_ref, rv_ref, ri_ref,
                 *, nb, bn):
    g = pl.program_id(0)

    @pl.when(g == 0)
    def _init():
        B = outf_ref.shape[0]
        rv_ref[...] = jnp.full((B, 8), _NEG, jnp.float32)
        ri_ref[...] = jnp.full((B, 8), -1, jnp.int32)

    q = outf_ref[...]
    B = q.shape[0]
    s = jax.lax.dot_general(q, bank_ref[...],
                            (((1,), (1,)), ((), ())),
                            preferred_element_type=jnp.float32)
    colg = g * bn + jax.lax.broadcasted_iota(jnp.int32, (B, bn), 1)
    s = jnp.where(mask_ref[...] > 0, _NEG, s)

    bvs, bis = [], []
    for _ in range(6):
        m = jnp.max(s, axis=1, keepdims=True)
        csel = jnp.min(jnp.where(s == m, colg, _BIG), axis=1, keepdims=True)
        bvs.append(m)
        bis.append(csel)
        s = jnp.where(colg == csel, _NEG, s)
    bv = jnp.concatenate(bvs, axis=1)
    bi = jnp.concatenate(bis, axis=1)

    pad2v = jnp.full((B, 2), _NEG, jnp.float32)
    pad2i = jnp.full((B, 2), -1, jnp.int32)
    cv = jnp.concatenate([rv_ref[...], bv, pad2v], axis=1)
    ci = jnp.concatenate([ri_ref[...], bi, pad2i], axis=1)
    nv, ni = _top6(cv, ci)
    rv_ref[...] = jnp.concatenate([nv, pad2v], axis=1)
    ri_ref[...] = jnp.concatenate([ni, pad2i], axis=1)

    @pl.when(g == nb - 1)
    def _finish():
        gc = jax.lax.dot_general(q, q,
                                 (((1,), (1,)), ((), ())),
                                 preferred_element_type=jnp.float32)
        winb = win_ref[...] > 0
        v = jnp.where(winb, gc, _NEG)
        colb = jax.lax.broadcasted_iota(jnp.int32, (B, B), 1)
        tir = tir_ref[...]
        pvs, pis = [], []
        for _ in range(6):
            m = jnp.max(v, axis=1, keepdims=True)
            bsel = jnp.min(jnp.where(v == m, colb, _BIG), axis=1, keepdims=True)
            jsel = jnp.max(jnp.where(colb == bsel, tir, -1), axis=1,
                           keepdims=True)
            pvs.append(m)
            pis.append(jsel)
            v = jnp.where(colb == bsel, _NEG, v)
        cv2 = jnp.concatenate([nv, jnp.concatenate(pvs, axis=1), pad2v, pad2v],
                              axis=1)
        ci2 = jnp.concatenate([ni, jnp.concatenate(pis, axis=1), pad2i, pad2i],
                              axis=1)
        fv, fi = _top6(cv2, ci2)
        vout_ref[...] = jnp.concatenate([fv, pad2v], axis=1)
        iout_ref[...] = jnp.concatenate([fi, pad2i], axis=1)


def _kl_body(g1_ref, if5_ref, qx_ref, tir_ref, win_ref, so_ref, disp_ref,
             out_ref, acc_ref, *, nchunk, b):
    c = pl.program_id(0)

    @pl.when(c == 0)
    def _init():
        acc_ref[0, 0] = 0.0

    if5c = if5_ref[...]
    tir = tir_ref[...]
    so = so_ref[...]
    e = ((if5c == tir) & (win_ref[...] > 0)).astype(jnp.float32)
    patch = jnp.dot(e, so, preferred_element_type=jnp.float32)
    hasp = jnp.sum(e, axis=1, keepdims=True) > 0
    g1w = g1_ref[...]
    cc = g1w.shape[1] // 2
    odd = (if5c - (if5c // 2) * 2) == 1
    base = jnp.where(odd, g1w[:, cc:], g1w[:, :cc])
    rows = jnp.where(hasp, patch, base)
    B = tir.shape[1]
    ch = if5c.shape[0]
    colb = jax.lax.broadcasted_iota(jnp.int32, (ch, B), 1)
    qoh = (qx_ref[...] == colb).astype(jnp.float32)
    soq = jnp.dot(qoh, so, preferred_element_type=jnp.float32)
    acc_ref[0, 0] += jnp.sum(rows * (jnp.log(rows) - soq))

    @pl.when(c == nchunk - 1)
    def _finish():
        out_ref[...] = jnp.reshape(acc_ref[0, 0] / float(b) + disp_ref[0, 0],
                                   (1, 1))


def _make_sc_gather(n, d, btot):
    info = plsc.get_sparse_core_info()
    nc, ns = info.num_cores, info.num_subcores
    nw = nc * ns
    bpw = btot // nw
    mesh = plsc.VectorSubcoreMesh(core_axis_name="c", subcore_axis_name="s")

    @functools.partial(
        pl.kernel, mesh=mesh,
        out_type=jax.ShapeDtypeStruct((btot, d), jnp.float32),
        scratch_types=[
            pltpu.VMEM((bpw,), jnp.int32),
            pltpu.VMEM((bpw, d), jnp.float32),
            pltpu.SemaphoreType.DMA,
        ],
    )
    def _sc_gather(table_hbm, idx_hbm, out_hbm, idx_v, rows_v, sem):
        wid = lax.axis_index("s") * nc + lax.axis_index("c")
        base = wid * bpw
        pltpu.sync_copy(idx_hbm.at[pl.ds(base, bpw)], idx_v)
        pltpu.async_copy(table_hbm.at[idx_v], rows_v, sem).wait()
        pltpu.sync_copy(rows_v, out_hbm.at[pl.ds(base, bpw)])

    return _sc_gather


def kernel(features, fea_bank, W_cls, b_cls, score_bank, trg_idx):
    B, D = features.shape
    N = fea_bank.shape[0]
    C = W_cls.shape[1]
    K = 5
    ti = trg_idx.astype(jnp.int32)
    tir = ti.reshape(1, B)
    tic = ti.reshape(B, 1)

    so, outf, win, disp = pl.pallas_call(
        _prep_body,
        out_shape=[
            jax.ShapeDtypeStruct((B, C), jnp.float32),
            jax.ShapeDtypeStruct((B, D), jnp.float32),
            jax.ShapeDtypeStruct((1, B), jnp.int32),
            jax.ShapeDtypeStruct((1, 1), jnp.float32),
        ],
    )(features, W_cls, b_cls.reshape(1, C), tir, tic)

    BN = 2048
    NB = -(-N // BN)
    npad = NB * BN

    maskp = jnp.zeros((1, npad), jnp.float32)
    maskp = maskp.at[0, N:].set(1.0)
    maskp = maskp.at[0, ti].set(1.0)

    vals6, idx6 = pl.pallas_call(
        functools.partial(_stream_body, nb=NB, bn=BN),
        grid=(NB,),
        in_specs=[
            pl.BlockSpec((B, D), lambda g: (0, 0)),
            pl.BlockSpec((BN, D), lambda g: (g, 0)),
            pl.BlockSpec((1, BN), lambda g: (0, g)),
            pl.BlockSpec((1, B), lambda g: (0, 0)),
            pl.BlockSpec((1, B), lambda g: (0, 0)),
        ],
        out_specs=[
            pl.BlockSpec((B, 8), lambda g: (0, 0)),
            pl.BlockSpec((B, 8), lambda g: (0, 0)),
        ],
        out_shape=[
            jax.ShapeDtypeStruct((B, 8), jnp.float32),
            jax.ShapeDtypeStruct((B, 8), jnp.int32),
        ],
        scratch_shapes=[
            pltpu.VMEM((B, 8), jnp.float32),
            pltpu.VMEM((B, 8), jnp.int32),
        ],
    )(outf, fea_bank, maskp, win, tir)

    if5 = idx6[:, 1:1 + K].reshape(B * K, 1)
    if5_flat = if5.reshape(B * K)

    sbw = score_bank.reshape(N // 2, 2 * C)
    g1 = _make_sc_gather(N // 2, 2 * C, B * K)(sbw, if5_flat // 2)

    CH = 640
    NCH = B * K // CH
    qx = (jnp.arange(B * K, dtype=jnp.int32) // K).reshape(B * K, 1)
    loss = pl.pallas_call(
        functools.partial(_kl_body, nchunk=NCH, b=B),
        grid=(NCH,),
        in_specs=[
            pl.BlockSpec((CH, 2 * C), lambda c: (c, 0)),
            pl.BlockSpec((CH, 1), lambda c: (c, 0)),
            pl.BlockSpec((CH, 1), lambda c: (c, 0)),
            pl.BlockSpec((1, B), lambda c: (0, 0)),
            pl.BlockSpec((1, B), lambda c: (0, 0)),
            pl.BlockSpec((B, C), lambda c: (0, 0)),
            pl.BlockSpec((1, 1), lambda c: (0, 0)),
        ],
        out_specs=pl.BlockSpec((1, 1), lambda c: (0, 0)),
        out_shape=jax.ShapeDtypeStruct((1, 1), jnp.float32),
        scratch_shapes=[pltpu.SMEM((1, 1), jnp.float32)],
    )(g1, if5, qx, tir, win, so, disp)

    return loss[0, 0]


# speedup vs baseline: 1.0740x; 1.0363x over previous
import functools

import jax
import jax.numpy as jnp
from jax.experimental import pallas as pl
from jax.experimental.pallas import tpu as pltpu
from jax.experimental.pallas import tpu_sc as plsc
from jax import lax

_NEG = float("-inf")
_BIG = 2**30


def _top6(cv, ci):
    vs, js = [], []
    for _ in range(6):
        m = jnp.max(cv, axis=1, keepdims=True)
        isel = jnp.min(jnp.where(cv == m, ci, _BIG), axis=1, keepdims=True)
        vs.append(m)
        js.append(isel)
        cv = jnp.where((cv == m) & (ci == isel), _NEG, cv)
    return jnp.concatenate(vs, axis=1), jnp.concatenate(js, axis=1)


def _prep_body(feat_ref, w_ref, b_ref, tir_ref, tic_ref,
               so_ref, outf_ref, win_ref, disp_ref):
    f = feat_ref[...]
    B = f.shape[0]
    preds = jnp.dot(f, w_ref[...], preferred_element_type=jnp.float32)
    preds = preds + b_ref[...]
    m = jnp.max(preds, axis=1, keepdims=True)
    e = jnp.exp(preds - m)
    so = e / jnp.sum(e, axis=1, keepdims=True)
    so_ref[...] = so

    nrm = jnp.sqrt(jnp.sum(f * f, axis=1, keepdims=True))
    nrm = jnp.maximum(nrm, 1e-12)
    outf_ref[...] = f / nrm

    sv = jnp.sum(so, axis=0, keepdims=True)
    disp = (jnp.sum(sv * sv) - jnp.sum(so * so)) / float(B)
    disp_ref[...] = jnp.reshape(disp, (1, 1))

    tir = tir_ref[...]
    tic = tic_ref[...]
    row = jax.lax.broadcasted_iota(jnp.int32, (B, B), 0)
    col = jax.lax.broadcasted_iota(jnp.int32, (B, B), 1)
    eq = (tic == tir) & (row > col)
    loser = jnp.max(jnp.where(eq, 1, 0), axis=0, keepdims=True)
    win_ref[...] = 1 - loser


def _stream_body(outf_ref, bank_ref, mask_ref,
                 win_ref, tir_ref, vout_ref, iout_ref, rv_ref, ri_ref,
                 *, nb, bn):
    g = pl.program_id(0)

    @pl.when(g == 0)
    def _init():
        B = outf_ref.shape[0]
        rv_ref[...] = jnp.full((B, 8), _NEG, jnp.float32)
        ri_ref[...] = jnp.full((B, 8), -1, jnp.int32)

    q = outf_ref[...]
    B = q.shape[0]
    s = jax.lax.dot_general(q, bank_ref[...],
                            (((1,), (1,)), ((), ())),
                            preferred_element_type=jnp.float32)
    colg = g * bn + jax.lax.broadcasted_iota(jnp.int32, (B, bn), 1)
    s = jnp.where(mask_ref[...] > 0, _NEG, s)

    bvs, bis = [], []
    for _ in range(6):
        m = jnp.max(s, axis=1, keepdims=True)
        csel = jnp.min(jnp.where(s == m, colg, _BIG), axis=1, keepdims=True)
        bvs.append(m)
        bis.append(csel)
        s = jnp.where(colg == csel, _NEG, s)
    bv = jnp.concatenate(bvs, axis=1)
    bi = jnp.concatenate(bis, axis=1)

    pad2v = jnp.full((B, 2), _NEG, jnp.float32)
    pad2i = jnp.full((B, 2), -1, jnp.int32)
    cv = jnp.concatenate([rv_ref[...], bv, pad2v], axis=1)
    ci = jnp.concatenate([ri_ref[...], bi, pad2i], axis=1)
    nv, ni = _top6(cv, ci)
    rv_ref[...] = jnp.concatenate([nv, pad2v], axis=1)
    ri_ref[...] = jnp.concatenate([ni, pad2i], axis=1)

    @pl.when(g == nb - 1)
    def _finish():
        gc = jax.lax.dot_general(q, q,
                                 (((1,), (1,)), ((), ())),
                                 preferred_element_type=jnp.float32)
        winb = win_ref[...] > 0
        v = jnp.where(winb, gc, _NEG)
        colb = jax.lax.broadcasted_iota(jnp.int32, (B, B), 1)
        tir = tir_ref[...]
        pvs, pis = [], []
        for _ in range(6):
            m = jnp.max(v, axis=1, keepdims=True)
            bsel = jnp.min(jnp.where(v == m, colb, _BIG), axis=1, keepdims=True)
            jsel = jnp.max(jnp.where(colb == bsel, tir, -1), axis=1,
                           keepdims=True)
            pvs.append(m)
            pis.append(jsel)
            v = jnp.where(colb == bsel, _NEG, v)
        cv2 = jnp.concatenate([nv, jnp.concatenate(pvs, axis=1), pad2v, pad2v],
                              axis=1)
        ci2 = jnp.concatenate([ni, jnp.concatenate(pis, axis=1), pad2i, pad2i],
                              axis=1)
        fv, fi = _top6(cv2, ci2)
        vout_ref[...] = jnp.concatenate([fv, pad2v], axis=1)
        iout_ref[...] = jnp.concatenate([fi, pad2i], axis=1)


def _kl_body(g1_ref, if5_ref, qx_ref, tir_ref, win_ref, so_ref, disp_ref,
             out_ref, acc_ref, *, nchunk, b):
    c = pl.program_id(0)

    @pl.when(c == 0)
    def _init():
        acc_ref[0, 0] = 0.0

    if5c = if5_ref[...]
    tir = tir_ref[...]
    so = so_ref[...]
    e = ((if5c == tir) & (win_ref[...] > 0)).astype(jnp.float32)
    patch = jnp.dot(e, so, preferred_element_type=jnp.float32)
    hasp = jnp.sum(e, axis=1, keepdims=True) > 0
    g1w = g1_ref[...]
    cc = g1w.shape[1] // 2
    odd = (if5c - (if5c // 2) * 2) == 1
    base = jnp.where(odd, g1w[:, cc:], g1w[:, :cc])
    rows = jnp.where(hasp, patch, base)
    B = tir.shape[1]
    ch = if5c.shape[0]
    colb = jax.lax.broadcasted_iota(jnp.int32, (ch, B), 1)
    qoh = (qx_ref[...] == colb).astype(jnp.float32)
    soq = jnp.dot(qoh, so, preferred_element_type=jnp.float32)
    acc_ref[0, 0] += jnp.sum(rows * (jnp.log(rows) - soq))

    @pl.when(c == nchunk - 1)
    def _finish():
        out_ref[...] = jnp.reshape(acc_ref[0, 0] / float(b) + disp_ref[0, 0],
                                   (1, 1))


def _make_sc_gather(n, d, btot):
    info = plsc.get_sparse_core_info()
    nc, ns = info.num_cores, info.num_subcores
    nw = nc * ns
    bpw = btot // nw
    mesh = plsc.VectorSubcoreMesh(core_axis_name="c", subcore_axis_name="s")

    @functools.partial(
        pl.kernel, mesh=mesh,
        out_type=jax.ShapeDtypeStruct((btot, d), jnp.float32),
        scratch_types=[
            pltpu.VMEM((bpw,), jnp.int32),
            pltpu.VMEM((bpw, d), jnp.float32),
            pltpu.SemaphoreType.DMA,
        ],
    )
    def _sc_gather(table_hbm, idx_hbm, out_hbm, idx_v, rows_v, sem):
        wid = lax.axis_index("s") * nc + lax.axis_index("c")
        base = wid * bpw
        pltpu.sync_copy(idx_hbm.at[pl.ds(base, bpw)], idx_v)
        pltpu.async_copy(table_hbm.at[idx_v], rows_v, sem).wait()
        pltpu.sync_copy(rows_v, out_hbm.at[pl.ds(base, bpw)])

    return _sc_gather


def kernel(features, fea_bank, W_cls, b_cls, score_bank, trg_idx):
    B, D = features.shape
    N = fea_bank.shape[0]
    C = W_cls.shape[1]
    K = 5
    ti = trg_idx.astype(jnp.int32)
    tir = ti.reshape(1, B)
    tic = ti.reshape(B, 1)

    so, outf, win, disp = pl.pallas_call(
        _prep_body,
        out_shape=[
            jax.ShapeDtypeStruct((B, C), jnp.float32),
            jax.ShapeDtypeStruct((B, D), jnp.float32),
            jax.ShapeDtypeStruct((1, B), jnp.int32),
            jax.ShapeDtypeStruct((1, 1), jnp.float32),
        ],
    )(features, W_cls, b_cls.reshape(1, C), tir, tic)

    BN = 2560
    NB = -(-N // BN)
    npad = NB * BN

    maskp = jnp.zeros((1, npad), jnp.float32)
    maskp = maskp.at[0, N:].set(1.0)
    maskp = maskp.at[0, ti].set(1.0)

    vals6, idx6 = pl.pallas_call(
        functools.partial(_stream_body, nb=NB, bn=BN),
        grid=(NB,),
        in_specs=[
            pl.BlockSpec((B, D), lambda g: (0, 0)),
            pl.BlockSpec((BN, D), lambda g: (g, 0)),
            pl.BlockSpec((1, BN), lambda g: (0, g)),
            pl.BlockSpec((1, B), lambda g: (0, 0)),
            pl.BlockSpec((1, B), lambda g: (0, 0)),
        ],
        out_specs=[
            pl.BlockSpec((B, 8), lambda g: (0, 0)),
            pl.BlockSpec((B, 8), lambda g: (0, 0)),
        ],
        out_shape=[
            jax.ShapeDtypeStruct((B, 8), jnp.float32),
            jax.ShapeDtypeStruct((B, 8), jnp.int32),
        ],
        scratch_shapes=[
            pltpu.VMEM((B, 8), jnp.float32),
            pltpu.VMEM((B, 8), jnp.int32),
        ],
    )(outf, fea_bank, maskp, win, tir)

    if5 = idx6[:, 1:1 + K].reshape(B * K, 1)
    if5_flat = if5.reshape(B * K)

    sbw = score_bank.reshape(N // 2, 2 * C)
    g1 = _make_sc_gather(N // 2, 2 * C, B * K)(sbw, if5_flat // 2)

    CH = 640
    NCH = B * K // CH
    qx = (jnp.arange(B * K, dtype=jnp.int32) // K).reshape(B * K, 1)
    loss = pl.pallas_call(
        functools.partial(_kl_body, nchunk=NCH, b=B),
        grid=(NCH,),
        in_specs=[
            pl.BlockSpec((CH, 2 * C), lambda c: (c, 0)),
            pl.BlockSpec((CH, 1), lambda c: (c, 0)),
            pl.BlockSpec((CH, 1), lambda c: (c, 0)),
            pl.BlockSpec((1, B), lambda c: (0, 0)),
            pl.BlockSpec((1, B), lambda c: (0, 0)),
            pl.BlockSpec((B, C), lambda c: (0, 0)),
            pl.BlockSpec((1, 1), lambda c: (0, 0)),
        ],
        out_specs=pl.BlockSpec((1, 1), lambda c: (0, 0)),
        out_shape=jax.ShapeDtypeStruct((1, 1), jnp.float32),
        scratch_shapes=[pltpu.SMEM((1, 1), jnp.float32)],
    )(g1, if5, qx, tir, win, so, disp)

    return loss[0, 0]
